# baseline (speedup 1.0000x reference)
import jax
import jax.numpy as jnp
from jax import lax
from jax.experimental import pallas as pl
from jax.experimental.pallas import tpu as pltpu
from jax.experimental.pallas import tpu_sc as plsc

D_MODEL = 64
SCALE = 8.0
NC = 2
NS = 16
NW = NC * NS
CHUNK = 128
PHYS = CHUNK // 2
LANES = 16
NBUF = 3
ROW_UNROLL = 16


def _emb_body(table_hbm, x_hbm, out_hbm,
              idx_v, pidx_v, rows_in, rows_out, gsem, osem):
    b_per_w = x_hbm.shape[0] // NW
    nchunks = b_per_w // CHUNK
    ngroups = nchunks // NBUF
    wid = lax.axis_index("s") * NC + lax.axis_index("c")
    base = wid * b_per_w
    pltpu.sync_copy(x_hbm.at[pl.ds(base, b_per_w)], idx_v)

    def prep_and_gather(b, off):
        @pl.loop(0, CHUNK, step=LANES)
        def _shift(i):
            pidx_v.at[b][pl.ds(i, LANES)] = (
                idx_v[pl.ds(off + i, LANES)] >> 1
            )

        pltpu.make_async_copy(
            table_hbm.at[pidx_v.at[b]], rows_in.at[b], gsem.at[b]
        ).start()

    def gather_wait(b):
        pltpu.make_async_copy(
            table_hbm.at[pidx_v.at[b]], rows_in.at[b], gsem.at[b]
        ).wait()

    def out_start(b, off):
        pltpu.make_async_copy(
            rows_out.at[b],
            out_hbm.at[pl.ds(pl.multiple_of((base + off) >> 1, PHYS), PHYS)],
            osem.at[b],
        ).start()

    def out_wait(b):
        pltpu.make_async_copy(
            rows_out.at[b], out_hbm.at[pl.ds(0, PHYS)], osem.at[b]
        ).wait()

    def scale_chunk(b, off):
        @pl.loop(0, CHUNK, step=ROW_UNROLL)
        def _rows(r0):
            p0 = r0 >> 1
            vids = idx_v[pl.ds(off + r0, ROW_UNROLL)]
            for dr in range(ROW_UNROLL):
                v = vids[dr]
                src = (v & 1) * D_MODEL
                pr = p0 + (dr >> 1)
                dst = (dr & 1) * D_MODEL
                for j in range(D_MODEL // LANES):
                    rows_out.at[b][pr, pl.ds(dst + j * LANES, LANES)] = (
                        rows_in.at[b][r0 + dr, pl.ds(src + j * LANES, LANES)]
                        * SCALE
                    )

    for b in range(NBUF):
        prep_and_gather(b, b * CHUNK)

    for b in range(NBUF):
        gather_wait(b)
        scale_chunk(b, b * CHUNK)
        prep_and_gather(b, (NBUF + b) * CHUNK)
        out_start(b, b * CHUNK)

    @pl.loop(1, ngroups)
    def _group(g):
        for b in range(NBUF):
            off = (g * NBUF + b) * CHUNK
            gather_wait(b)
            out_wait(b)
            scale_chunk(b, off)

            @pl.when(g < ngroups - 1)
            def _():
                prep_and_gather(b, off + NBUF * CHUNK)

            out_start(b, off)

    for b in range(NBUF):
        out_wait(b)


def kernel(x, table):
    B = x.shape[0] * x.shape[1]
    xf = x.reshape(B)
    tablev = table.reshape(table.shape[0] // 2, 2 * table.shape[1])
    b_per_w = B // NW
    mesh = plsc.VectorSubcoreMesh(core_axis_name="c", subcore_axis_name="s")
    run = pl.kernel(
        _emb_body,
        out_type=jax.ShapeDtypeStruct((B // 2, 2 * D_MODEL), jnp.float32),
        mesh=mesh,
        scratch_types=[
            pltpu.VMEM((b_per_w,), jnp.int32),
            pltpu.VMEM((NBUF, CHUNK), jnp.int32),
            pltpu.VMEM((NBUF, CHUNK, 2 * D_MODEL), jnp.float32),
            pltpu.VMEM((NBUF, PHYS, 2 * D_MODEL), jnp.float32),
            pltpu.SemaphoreType.DMA((NBUF,)),
            pltpu.SemaphoreType.DMA((NBUF,)),
        ],
    )
    out = run(tablev, xf)
    return out.reshape(x.shape[0], x.shape[1], D_MODEL)


# speedup vs baseline: 1.2861x; 1.2861x over previous
import jax
import jax.numpy as jnp
from jax import lax
from jax.experimental import pallas as pl
from jax.experimental.pallas import tpu as pltpu
from jax.experimental.pallas import tpu_sc as plsc

D_MODEL = 64
SCALE = 8.0
NC = 2
NS = 16
NW = NC * NS
SEQ = 200
SPLITS = ((0, 104), (104, 96))
LANES = 16
NBUF = 4
NIBUF = 2 * NBUF
ROW_UNROLL = 8


def _emb_body(table_hbm, x_hbm, out_hbm, idx_v, rows_in, rows_out,
              gsem, osem, isem):
    rows_per_w = x_hbm.shape[0] // NW
    wid = lax.axis_index("s") * NC + lax.axis_index("c")
    rbase = wid * rows_per_w

    def idx_start(slot, row):
        pltpu.make_async_copy(
            x_hbm.at[rbase + row], idx_v.at[slot], isem.at[slot]
        ).start()

    def idx_wait(slot):
        pltpu.make_async_copy(
            x_hbm.at[rbase], idx_v.at[slot], isem.at[slot]
        ).wait()

    def gather_start(b, slot):
        for off, n in SPLITS:
            pltpu.make_async_copy(
                table_hbm.at[idx_v.at[slot].at[pl.ds(off, n)]],
                rows_in.at[b].at[pl.ds(off, n)],
                gsem.at[b],
            ).start()

    def gather_wait(b):
        for off, n in SPLITS:
            pltpu.make_async_copy(
                table_hbm.at[idx_v.at[0].at[pl.ds(off, n)]],
                rows_in.at[b].at[pl.ds(off, n)],
                gsem.at[b],
            ).wait()

    def out_start(b, row):
        pltpu.make_async_copy(
            rows_out.at[b], out_hbm.at[rbase + row], osem.at[b]
        ).start()

    def out_wait(b):
        pltpu.make_async_copy(
            rows_out.at[b], out_hbm.at[rbase], osem.at[b]
        ).wait()

    def scale(b):
        @pl.loop(0, SEQ, step=ROW_UNROLL)
        def _rows(r0):
            for dr in range(ROW_UNROLL):
                for j in range(D_MODEL // LANES):
                    sl = (r0 + dr, pl.ds(j * LANES, LANES))
                    rows_out.at[b][sl] = rows_in.at[b][sl] * SCALE

    for j in range(NIBUF):
        idx_start(j, j)
    for j in range(NBUF):
        idx_wait(j)
        gather_start(j, j)

    for b in range(NBUF):
        row = b
        gather_wait(b)
        scale(b)
        idx_wait((row + NBUF) % NIBUF)
        gather_start(b, (row + NBUF) % NIBUF)
        idx_start(row % NIBUF, row + NIBUF)
        out_start(b, row)

    ngroups = rows_per_w // NBUF

    @pl.loop(1, ngroups)
    def _group(g):
        for b in range(NBUF):
            row = g * NBUF + b
            gather_wait(b)
            out_wait(b)
            scale(b)

            @pl.when(g < ngroups - 1)
            def _():
                idx_wait((row + NBUF) % NIBUF)
                gather_start(b, (row + NBUF) % NIBUF)

            @pl.when(row + NIBUF < rows_per_w)
            def _():
                idx_start(row % NIBUF, row + NIBUF)

            out_start(b, row)

    for b in range(NBUF):
        out_wait(b)


def kernel(x, table):
    rows_per_w = x.shape[0] // NW
    assert x.shape[0] % NW == 0 and rows_per_w % NBUF == 0
    mesh = plsc.VectorSubcoreMesh(core_axis_name="c", subcore_axis_name="s")
    run = pl.kernel(
        _emb_body,
        out_type=jax.ShapeDtypeStruct((x.shape[0], x.shape[1], D_MODEL),
                                      jnp.float32),
        mesh=mesh,
        compiler_params=pltpu.CompilerParams(use_tc_tiling_on_sc=False),
        scratch_types=[
            pltpu.VMEM((NIBUF, SEQ), jnp.int32),
            pltpu.VMEM((NBUF, SEQ, D_MODEL), jnp.float32),
            pltpu.VMEM((NBUF, SEQ, D_MODEL), jnp.float32),
            pltpu.SemaphoreType.DMA((NBUF,)),
            pltpu.SemaphoreType.DMA((NBUF,)),
            pltpu.SemaphoreType.DMA((NIBUF,)),
        ],
    )
    return run(table, x)


# speedup vs baseline: 1.5842x; 1.2318x over previous
import functools

import jax
import jax.numpy as jnp
from jax import lax
from jax.experimental import pallas as pl
from jax.experimental.pallas import tpu as pltpu
from jax.experimental.pallas import tpu_sc as plsc

D_MODEL = 64
SCALE = 8.0
NC = 2
NS = 16
NW = NC * NS
SEQ = 200
SPLITS = ((0, 104), (104, 96))
NBUF = 4
LEAD = 2
FMT_BLOCK = 2048


def _fmt_body(t_ref, o_ref):
    t8 = (t_ref[...] * SCALE).T
    o_ref[...] = jnp.concatenate([t8, t8], axis=1)


def _format_table(table_t):
    vocab = table_t.shape[1]
    return pl.pallas_call(
        _fmt_body,
        grid=(pl.cdiv(vocab, FMT_BLOCK),),
        in_specs=[pl.BlockSpec((D_MODEL, FMT_BLOCK), lambda i: (0, i))],
        out_specs=pl.BlockSpec((FMT_BLOCK, 2 * D_MODEL), lambda i: (i, 0)),
        out_shape=jax.ShapeDtypeStruct((vocab, 2 * D_MODEL), jnp.float32),
        compiler_params=pltpu.CompilerParams(
            dimension_semantics=("arbitrary",)),
    )(table_t)


def _gather_body(table_hbm, x_hbm, out_hbm, idx_v, rows, gsem, osem, isem):
    rows_per_w = x_hbm.shape[0] // NW
    wid = lax.axis_index("s") * NC + lax.axis_index("c")
    rbase = wid * rows_per_w

    def idx_start(slot, row):
        pltpu.make_async_copy(
            x_hbm.at[rbase + row], idx_v.at[slot], isem.at[slot]
        ).start()

    def idx_wait(slot):
        pltpu.make_async_copy(
            x_hbm.at[rbase], idx_v.at[slot], isem.at[slot]
        ).wait()

    def gather_start(b, slot):
        for off, n in SPLITS:
            pltpu.make_async_copy(
                table_hbm.at[idx_v.at[slot].at[pl.ds(off, n)]],
                rows.at[b].at[pl.ds(off, n)],
                gsem.at[b],
            ).start()

    def gather_wait(b):
        for off, n in SPLITS:
            pltpu.make_async_copy(
                table_hbm.at[idx_v.at[0].at[pl.ds(off, n)]],
                rows.at[b].at[pl.ds(off, n)],
                gsem.at[b],
            ).wait()

    def out_start(b, row):
        pltpu.make_async_copy(
            rows.at[b],
            out_hbm.at[pl.ds((rbase + row) * SEQ, SEQ)],
            osem.at[b],
        ).start()

    def out_wait(b):
        pltpu.make_async_copy(
            rows.at[b], out_hbm.at[pl.ds(0, SEQ)], osem.at[b]
        ).wait()

    for j in range(NBUF):
        idx_start(j, j)
    for j in range(LEAD):
        idx_wait(j)
        gather_start(j, j)

    for i in range(NBUF):
        b, s = i, (i + LEAD) % NBUF
        gather_wait(b)
        out_start(b, i)
        idx_start(b, i + NBUF)
        if i >= LEAD:
            out_wait(s)
        idx_wait(s)
        gather_start(s, s)

    @pl.loop(1, rows_per_w // NBUF)
    def _group(g):
        for b in range(NBUF):
            row = g * NBUF + b
            s = (b + LEAD) % NBUF
            gather_wait(b)
            out_start(b, row)

            @pl.when(row + NBUF < rows_per_w)
            def _():
                idx_start(b, row + NBUF)

            @pl.when(row + LEAD < rows_per_w)
            def _():
                out_wait(s)
                idx_wait(s)
                gather_start(s, s)

    for b in range(NBUF):
        out_wait(b)


def _sc_gather(table_f, x):
    n_tok = x.shape[0] * x.shape[1]
    mesh = plsc.VectorSubcoreMesh(core_axis_name="c", subcore_axis_name="s")
    run = pl.kernel(
        _gather_body,
        out_type=jax.ShapeDtypeStruct((n_tok, 2 * D_MODEL), jnp.float32),
        mesh=mesh,
        compiler_params=pltpu.CompilerParams(use_tc_tiling_on_sc=False),
        scratch_types=[
            pltpu.VMEM((NBUF, SEQ), jnp.int32),
            pltpu.VMEM((NBUF, SEQ, 2 * D_MODEL), jnp.float32),
            pltpu.SemaphoreType.DMA((NBUF,)),
            pltpu.SemaphoreType.DMA((NBUF,)),
            pltpu.SemaphoreType.DMA((NBUF,)),
        ],
    )
    return run(table_f, x)


def kernel(x, table):
    table_f = _format_table(table.T)
    out = _sc_gather(table_f, x)
    return out[:, :D_MODEL].reshape(x.shape[0], x.shape[1], D_MODEL)
